# Initial kernel scaffold; baseline (speedup 1.0000x reference)
import jax
import jax.numpy as jnp
from jax.experimental import pallas as pl


def kernel(text, table, W, b):
    raise NotImplementedError("write your pallas kernel here")


# speedup vs baseline: 2.0602x; 2.0602x over previous
import functools

import jax
import jax.numpy as jnp
from jax import lax
from jax.experimental import pallas as pl
from jax.experimental.pallas import tpu as pltpu
from jax.experimental.pallas import tpu_sc as plsc

_NC = 2
_NS = 16
_NW = _NC * _NS
_L = 16


def _pooled_sum_sc(text_flat, table, B, S, D, CH):
    R = CH * S
    elems_per_w = B // _NW
    mesh = plsc.VectorSubcoreMesh(core_axis_name="c", subcore_axis_name="s")

    @functools.partial(
        pl.kernel,
        out_type=jax.ShapeDtypeStruct((B, D), jnp.float32),
        mesh=mesh,
        scratch_types=[
            pltpu.VMEM((R,), jnp.int32),
            pltpu.VMEM((R, D), jnp.float32),
            pltpu.VMEM((CH, D), jnp.float32),
            pltpu.SemaphoreType.DMA,
        ],
        compiler_params=pltpu.CompilerParams(use_tc_tiling_on_sc=False),
    )
    def k(text_hbm, table_hbm, out_hbm, idx_v, rows_v, acc_v, sem):
        wid = lax.axis_index("s") * _NC + lax.axis_index("c")

        @pl.loop(0, elems_per_w // CH)
        def _(c):
            b_base = wid * elems_per_w + c * CH
            pltpu.sync_copy(text_hbm.at[pl.ds(b_base * S, R)], idx_v)
            pltpu.async_copy(table_hbm.at[idx_v], rows_v, sem).wait()
            for j in range(CH):
                def red(i, carry, j=j):
                    a0, a1 = carry
                    r = j * S + i
                    return (a0 + rows_v[r, pl.ds(0, _L)],
                            a1 + rows_v[r, pl.ds(_L, _L)])
                a0, a1 = lax.fori_loop(
                    0, S, red,
                    (jnp.zeros((_L,), jnp.float32),
                     jnp.zeros((_L,), jnp.float32)))
                acc_v[j, pl.ds(0, _L)] = a0
                acc_v[j, pl.ds(_L, _L)] = a1
            pltpu.sync_copy(acc_v, out_hbm.at[pl.ds(b_base, CH)])

    return k(text_flat, table)


def _linear_tc(pooled_sum, Wt, b2d, inv_s):
    B, D = pooled_sum.shape
    C = Wt.shape[1]

    def body(p_ref, w_ref, b_ref, o_ref):
        p = p_ref[...] * inv_s
        o_ref[...] = jnp.dot(
            p, w_ref[...], preferred_element_type=jnp.float32) + b_ref[...]

    return pl.pallas_call(
        body,
        out_shape=jax.ShapeDtypeStruct((B, C), jnp.float32),
    )(pooled_sum, Wt, b2d)


def kernel(text, table, W, b):
    B, S = text.shape
    V, D = table.shape
    C = W.shape[0]
    text_flat = text.reshape(-1).astype(jnp.int32)
    pooled_sum = _pooled_sum_sc(text_flat, table, B, S, D, CH=8)
    return _linear_tc(pooled_sum, W.T, b.reshape(1, C), 1.0 / S)


# speedup vs baseline: 2.3020x; 1.1173x over previous
import functools

import jax
import jax.numpy as jnp
from jax import lax
from jax.experimental import pallas as pl
from jax.experimental.pallas import tpu as pltpu
from jax.experimental.pallas import tpu_sc as plsc

_NC = 2
_NS = 16
_NW = _NC * _NS
_L = 16


def _pooled_sum_sc(text_flat, table, B, S, D, CH):
    R = CH * S
    elems_per_w = B // _NW
    n_chunks = elems_per_w // CH
    assert n_chunks % 2 == 0 and S % 2 == 0
    mesh = plsc.VectorSubcoreMesh(core_axis_name="c", subcore_axis_name="s")

    @functools.partial(
        pl.kernel,
        out_type=jax.ShapeDtypeStruct((B, D), jnp.float32),
        mesh=mesh,
        scratch_types=[
            pltpu.VMEM((R,), jnp.int32),
            pltpu.VMEM((R,), jnp.int32),
            pltpu.VMEM((R, D), jnp.float32),
            pltpu.VMEM((R, D), jnp.float32),
            pltpu.VMEM((CH, D), jnp.float32),
            pltpu.SemaphoreType.DMA,
            pltpu.SemaphoreType.DMA,
        ],
        compiler_params=pltpu.CompilerParams(use_tc_tiling_on_sc=False),
    )
    def k(text_hbm, table_hbm, out_hbm,
          idx_v0, idx_v1, rows_v0, rows_v1, acc_v, sem0, sem1):
        wid = lax.axis_index("s") * _NC + lax.axis_index("c")
        base_pos = wid * elems_per_w * S
        idx_b = (idx_v0, idx_v1)
        row_b = (rows_v0, rows_v1)
        sem_b = (sem0, sem1)

        def issue(c, buf):
            pltpu.sync_copy(text_hbm.at[pl.ds(base_pos + c * R, R)], idx_b[buf])
            pltpu.async_copy(table_hbm.at[idx_b[buf]], row_b[buf], sem_b[buf])

        def wait_gather(buf):
            pltpu.make_async_copy(
                table_hbm.at[idx_b[buf]], row_b[buf], sem_b[buf]).wait()

        def reduce_out(c, buf):
            rows_v = row_b[buf]
            z = jnp.zeros((_L,), jnp.float32)
            for j in range(CH):
                @plsc.parallel_loop(0, S, step=2, unroll=2,
                                    carry=(z, z, z, z))
                def red(i, carry, j=j):
                    a0, a1, b0, b1 = carry
                    r = j * S + i
                    return (a0 + rows_v[r, pl.ds(0, _L)],
                            a1 + rows_v[r, pl.ds(_L, _L)],
                            b0 + rows_v[r + 1, pl.ds(0, _L)],
                            b1 + rows_v[r + 1, pl.ds(_L, _L)])
                a0, a1, b0, b1 = red
                acc_v[j, pl.ds(0, _L)] = a0 + b0
                acc_v[j, pl.ds(_L, _L)] = a1 + b1
            pltpu.sync_copy(
                acc_v, out_hbm.at[pl.ds(wid * elems_per_w + c * CH, CH)])

        issue(0, 0)

        @pl.loop(0, n_chunks // 2)
        def _(h):
            c0 = 2 * h
            wait_gather(0)
            issue(c0 + 1, 1)
            reduce_out(c0, 0)
            wait_gather(1)

            @pl.when(h + 1 < n_chunks // 2)
            def _():
                issue(c0 + 2, 0)

            reduce_out(c0 + 1, 1)

    return k(text_flat, table)


def _linear_tc(pooled_sum, Wt, b2d, inv_s):
    B, D = pooled_sum.shape
    C = Wt.shape[1]

    def body(p_ref, w_ref, b_ref, o_ref):
        p = p_ref[...] * inv_s
        o_ref[...] = jnp.dot(
            p, w_ref[...], preferred_element_type=jnp.float32) + b_ref[...]

    return pl.pallas_call(
        body,
        out_shape=jax.ShapeDtypeStruct((B, C), jnp.float32),
    )(pooled_sum, Wt, b2d)


def kernel(text, table, W, b):
    B, S = text.shape
    V, D = table.shape
    C = W.shape[0]
    text_flat = text.reshape(-1).astype(jnp.int32)
    pooled_sum = _pooled_sum_sc(text_flat, table, B, S, D, CH=8)
    return _linear_tc(pooled_sum, W.T, b.reshape(1, C), 1.0 / S)
